# Initial kernel scaffold; baseline (speedup 1.0000x reference)
import jax
import jax.numpy as jnp
from jax.experimental import pallas as pl


def kernel(x, edge_index, edge_attr, params):
    raise NotImplementedError("write your pallas kernel here")


# speedup vs baseline: 17.4433x; 17.4433x over previous
import functools

import jax
import jax.numpy as jnp
from jax import lax
from jax.experimental import pallas as pl
from jax.experimental.pallas import tpu as pltpu
from jax.experimental.pallas import tpu_sc as plsc

_N = 10000
_E = 320000
_F_IN = 128
_NUM_CLASSES = 5
_LAYERS = [(128, 32, 4, True), (128, 32, 4, True), (128, 64, 4, True),
           (256, 64, 4, True), (256, 32, 4, True), (128, 32, 1, False)]
_RES = [(128, 128), None, (128, 256), None, (256, 128), (128, 32)]

_CH = 128
_NW = 32
_EPAD = 323584
_CPW = _EPAD // (_CH * _NW)
_RB = 1000
_GRID = _N // _RB
_NPAD = 10240
_NPS = _NPAD // 16


@functools.lru_cache(maxsize=None)
def _sc_edge_kernel(w_row, h, c, head_off, has_ones):
    hc0 = w_row - 16 if has_ones else w_row
    nheads = hc0 // c
    nv = w_row // 16

    mesh = plsc.VectorSubcoreMesh(core_axis_name="c", subcore_axis_name="s")

    @functools.partial(
        pl.kernel,
        out_type=jax.ShapeDtypeStruct((2 * _NPAD, w_row), jnp.float32),
        mesh=mesh,
        compiler_params=pltpu.CompilerParams(needs_layout_passes=False,
                                             use_tc_tiling_on_sc=False),
        scratch_types=[
            pltpu.VMEM_SHARED((_NPAD, w_row), jnp.float32),
            pltpu.VMEM((_CH,), jnp.int32),
            pltpu.VMEM((_CH,), jnp.int32),
            pltpu.VMEM((_CH, 16), jnp.float32),
            pltpu.VMEM((_CH, 16), jnp.float32),
            pltpu.VMEM((_CH, h), jnp.float32),
            pltpu.VMEM((_CH * h,), jnp.float32),
            pltpu.VMEM((_CH, w_row), jnp.float32),
            pltpu.SemaphoreType.DMA,
            pltpu.SemaphoreType.DMA,
            pltpu.SemaphoreType.DMA,
        ],
    )
    def k(xp_hbm, asrc_hbm, adst_hbm, aedg_hbm, src_hbm, dst_hbm, out_hbm,
          acc, src_v, dst_v, asg_v, adg_v, aedg_v, w_v, rows_v,
          sem0, sem1, sem2):
        cid = lax.axis_index("c")
        sid = lax.axis_index("s")
        wid = cid * 16 + sid
        zeros16 = jnp.zeros((16,), jnp.float32)
        lanes = lax.iota(jnp.int32, 16)

        def zrow(i, carry):
            for j in range(nv):
                rows_v[i, pl.ds(16 * j, 16)] = zeros16
            return carry
        lax.fori_loop(0, _CH, zrow, 0)
        for rep in range(_NPS // _CH):
            pltpu.sync_copy(rows_v,
                            acc.at[pl.ds(sid * _NPS + rep * _CH, _CH)])
        plsc.subcore_barrier()

        w_lo = 0 if has_ones else head_off
        w_hi = h if has_ones else head_off + nheads

        def chunk(t, carry):
            base = (wid * _CPW + t) * _CH
            pltpu.sync_copy(src_hbm.at[pl.ds(base, _CH)], src_v)
            pltpu.sync_copy(dst_hbm.at[pl.ds(base, _CH)], dst_v)
            pltpu.sync_copy(aedg_hbm.at[pl.ds(base, _CH)], aedg_v)
            cps = [pltpu.async_copy(asrc_hbm.at[src_v], asg_v, sem0),
                   pltpu.async_copy(adst_hbm.at[dst_v], adg_v, sem1),
                   pltpu.async_copy(xp_hbm.at[src_v], rows_v, sem2)]
            for cp in cps:
                cp.wait()

            for g in range(_CH // 16):
                eidx = lanes + (16 * g)
                for hh in range(w_lo, w_hi):
                    col = jnp.full((16,), hh, jnp.int32)
                    al = (plsc.load_gather(asg_v, [eidx, col])
                          + plsc.load_gather(adg_v, [eidx, col])
                          + plsc.load_gather(aedg_v, [eidx, col]))
                    al = jnp.maximum(al, 0.2 * al)
                    plsc.store_scatter(w_v, [eidx * h + hh], jnp.exp(al))

            def scale(e, carry):
                e16 = jnp.full((16,), e * h, jnp.int32)
                for head in range(nheads):
                    wb = plsc.load_gather(w_v, [e16 + (head_off + head)])
                    for j in range(head * c // 16, (head + 1) * c // 16):
                        rows_v[e, pl.ds(16 * j, 16)] = (
                            rows_v[e, pl.ds(16 * j, 16)] * wb)
                if has_ones:
                    wl = plsc.load_gather(
                        w_v, [e16 + jnp.minimum(lanes, h - 1)])
                    rows_v[e, pl.ds(hc0, 16)] = (
                        rows_v[e, pl.ds(hc0, 16)] * wl)
                return carry
            lax.fori_loop(0, _CH, scale, 0)

            pltpu.sync_copy(rows_v, acc.at[dst_v], add=True)
            return carry
        lax.fori_loop(0, _CPW, chunk, 0)

        plsc.subcore_barrier()
        pltpu.sync_copy(acc.at[pl.ds(sid * _NPS, _NPS)],
                        out_hbm.at[pl.ds(cid * _NPAD + sid * _NPS, _NPS)])

    return k


def _full_spec(shape):
    return pl.BlockSpec(shape, lambda i: tuple(0 for _ in shape))


def _row_spec(cols):
    return pl.BlockSpec((_RB, cols), lambda i: (i, 0))


def _stat_spec(cols):
    return pl.BlockSpec((1, cols), lambda i: (0, 0))


def _aedg_call(edge_attr_pad, we_list, ae_list):
    eb = 2048
    grid = _EPAD // eb
    hs = [1 if w.shape[1] == 32 else 4 for w in we_list]

    def body(ea_ref, *refs):
        wrefs = refs[:6]
        arefs = refs[6:12]
        orefs = refs[12:]
        i = pl.program_id(0)
        row = i * eb + lax.broadcasted_iota(jnp.int32, (eb, 1), 0)
        pad = jnp.where(row >= _E, -1e30, 0.0)
        ea = ea_ref[...]
        for li in range(6):
            hc = wrefs[li].shape[1]
            h = hs[li]
            c = hc // h
            ep = jnp.dot(ea, wrefs[li][...],
                         preferred_element_type=jnp.float32)
            m = ep * arefs[li][...]
            cols = [jnp.sum(m[:, hh * c:(hh + 1) * c], axis=1, keepdims=True)
                    for hh in range(h)]
            a = jnp.concatenate(cols, axis=1) if h > 1 else cols[0]
            orefs[li][...] = a + pad

    in_specs = [pl.BlockSpec((eb, 3), lambda i: (i, 0))]
    in_specs += [_full_spec(w.shape) for w in we_list]
    in_specs += [_stat_spec(a.shape[1]) for a in ae_list]
    out_specs = [pl.BlockSpec((eb, h), lambda i: (i, 0)) for h in hs]
    out_shape = [jax.ShapeDtypeStruct((_EPAD, h), jnp.float32) for h in hs]
    return pl.pallas_call(
        body, grid=(grid,), in_specs=in_specs, out_specs=out_specs,
        out_shape=out_shape)(edge_attr_pad, *we_list, *ae_list)


def _ones_pat(h):
    return (lax.broadcasted_iota(jnp.int32, (1, 16), 1) < h).astype(jnp.float32)


def _emit_slabs(xp, h, c, slab_refs):
    hc = h * c
    hc0 = min(hc, 128)
    pat = jnp.broadcast_to(_ones_pat(h), (_RB, 16))
    slab_refs[0][...] = jnp.concatenate([xp[:, :hc0], pat], axis=1)
    if hc > 128:
        slab_refs[1][...] = xp[:, 128:]


def _prep_outputs(li):
    fin, c, h, concat = _LAYERS[li]
    hc = h * c
    hc0 = min(hc, 128)
    shapes = [jax.ShapeDtypeStruct((_N, hc0 + 16), jnp.float32)]
    specs = [_row_spec(hc0 + 16)]
    if hc > 128:
        shapes.append(jax.ShapeDtypeStruct((_N, 128), jnp.float32))
        specs.append(_row_spec(128))
    shapes += [jax.ShapeDtypeStruct((_N, 16), jnp.float32)] * 2
    specs += [_row_spec(16)] * 2
    fo = _RES[li][1] if _RES[li] is not None else _LAYERS[li][0]
    shapes.append(jax.ShapeDtypeStruct((_N, fo), jnp.float32))
    specs.append(_row_spec(fo))
    return shapes, specs


def _entry_call(x, w0, as0, ad0, res_w, res_b):
    def body(x_ref, w_ref, as_ref, ad_ref, rw_ref, rb_ref,
             slab_ref, asrc_ref, adst_ref, idt_ref):
        xb = x_ref[...]
        xp = jnp.dot(xb, w_ref[...], preferred_element_type=jnp.float32)
        _emit_slabs(xp, 4, 32, [slab_ref])
        _emit_tables(xp, as_ref[...], ad_ref[...], 4, 32, asrc_ref, adst_ref)
        idt_ref[...] = (jnp.dot(xb, rw_ref[...],
                                preferred_element_type=jnp.float32)
                        + rb_ref[...])

    shapes, specs = _prep_outputs(0)
    in_specs = [_row_spec(_F_IN), _full_spec(w0.shape), _stat_spec(128),
                _stat_spec(128), _full_spec(res_w.shape),
                pl.BlockSpec((1, res_b.shape[1]), lambda i: (0, 0))]
    return pl.pallas_call(
        body, grid=(_GRID,), in_specs=in_specs, out_specs=specs,
        out_shape=shapes)(x, w0, as0, ad0, res_w, res_b)


def _combine_call(li, partials, bias):
    fin, c, h, concat = _LAYERS[li]
    hc = h * c
    hc0 = min(hc, 128)

    flat = []
    for p in partials:
        flat += [p[:_N], p[_NPAD:_NPAD + _N]]

    def body(*refs):
        i = pl.program_id(0)
        n_in = len(flat) + 1
        prefs = refs[:len(flat)]
        bias_ref = refs[len(flat)]
        out_ref, cs_ref, css_ref = refs[n_in:]
        m0 = prefs[0][...] + prefs[1][...]
        den = m0[:, hc0:hc0 + h] + 1e-16
        feats = m0[:, :hc0]
        if len(prefs) == 4:
            feats = jnp.concatenate([feats, prefs[2][...] + prefs[3][...]],
                                    axis=1)
        cols = []
        for hh in range(h):
            cols.append(feats[:, hh * c:(hh + 1) * c] / den[:, hh:hh + 1])
        out = jnp.concatenate(cols, axis=1) + bias_ref[...]
        out_ref[...] = out

        @pl.when(i == 0)
        def _():
            cs_ref[...] = jnp.zeros_like(cs_ref)
            css_ref[...] = jnp.zeros_like(css_ref)
        cs_ref[...] += jnp.sum(out, axis=0, keepdims=True)
        css_ref[...] += jnp.sum(out * out, axis=0, keepdims=True)

    in_specs = [_row_spec(p.shape[1]) for p in flat]
    in_specs.append(_stat_spec(hc))
    out_specs = [_row_spec(hc), _stat_spec(hc), _stat_spec(hc)]
    out_shape = [jax.ShapeDtypeStruct((_N, hc), jnp.float32),
                 jax.ShapeDtypeStruct((1, hc), jnp.float32),
                 jax.ShapeDtypeStruct((1, hc), jnp.float32)]
    return pl.pallas_call(
        body, grid=(_GRID,), in_specs=in_specs, out_specs=out_specs,
        out_shape=out_shape)(*flat, bias.reshape(1, hc))


def _bn_relu(z, cs, css, g, b, idt):
    mu = cs * (1.0 / _N)
    var = css * (1.0 / _N) - mu * mu
    bn = g * (z - mu) / jnp.sqrt(var + 1e-5) + b
    if idt is not None:
        bn = bn + idt
    return jnp.maximum(bn, 0.0)


def _fuse_call(li, out, cs, css, g, b, idt, params):
    nl = li + 1
    fin, c, h, concat = _LAYERS[nl]
    hc = h * c
    w_next = params[f"conv{nl}_W"]
    a_s = params[f"conv{nl}_as"].reshape(1, hc)
    a_d = params[f"conv{nl}_ad"].reshape(1, hc)
    res = _RES[nl]
    hc_prev = out.shape[1]
    if res is not None:
        rw = params[f"res{nl}_W"]
        rb = params[f"res{nl}_b"].reshape(1, -1)

    def body(*refs):
        (out_ref, cs_ref, css_ref, g_ref, b_ref, idt_ref, w_ref,
         as_ref, ad_ref) = refs[:9]
        rest = refs[9:]
        if res is not None:
            rw_ref, rb_ref = rest[:2]
            orefs = rest[2:]
        else:
            orefs = rest
        hnew = _bn_relu(out_ref[...], cs_ref[...], css_ref[...],
                        g_ref[...], b_ref[...], idt_ref[...])
        xp = jnp.dot(hnew, w_ref[...], preferred_element_type=jnp.float32)
        nslab = 2 if hc > 128 else 1
        _emit_slabs(xp, h, c, list(orefs[:nslab]))
        _emit_tables(xp, as_ref[...], ad_ref[...], h, c,
                     orefs[nslab], orefs[nslab + 1])
        if res is not None:
            orefs[nslab + 2][...] = (
                jnp.dot(hnew, rw_ref[...], preferred_element_type=jnp.float32)
                + rb_ref[...])
        else:
            orefs[nslab + 2][...] = hnew

    shapes, specs = _prep_outputs(nl)
    in_specs = [_row_spec(hc_prev), _stat_spec(hc_prev), _stat_spec(hc_prev),
                _stat_spec(hc_prev), _stat_spec(hc_prev), _row_spec(hc_prev),
                _full_spec(w_next.shape), _stat_spec(hc), _stat_spec(hc)]
    args = [out, cs, css, g.reshape(1, -1), b.reshape(1, -1), idt,
            w_next, a_s, a_d]
    if res is not None:
        in_specs += [_full_spec(rw.shape), _stat_spec(rb.shape[1])]
        args += [rw, rb]
    return pl.pallas_call(
        body, grid=(_GRID,), in_specs=in_specs, out_specs=specs,
        out_shape=shapes)(*args)


def _dense_stats_call(z, cs, css, g, b, idt, w, bias):
    cols_in = z.shape[1]
    cols_out = w.shape[1]
    has_idt = idt is not None

    def body(*refs):
        i = pl.program_id(0)
        idx = 0
        z_ref = refs[0]; cs_ref = refs[1]; css_ref = refs[2]
        g_ref = refs[3]; b_ref = refs[4]
        idx = 5
        idt_ref = None
        if has_idt:
            idt_ref = refs[idx]; idx += 1
        w_ref = refs[idx]; bias_ref = refs[idx + 1]
        zo_ref, ocs_ref, ocss_ref = refs[idx + 2:]
        hnew = _bn_relu(z_ref[...], cs_ref[...], css_ref[...],
                        g_ref[...], b_ref[...],
                        idt_ref[...] if has_idt else None)
        zo = (jnp.dot(hnew, w_ref[...], preferred_element_type=jnp.float32)
              + bias_ref[...])
        zo_ref[...] = zo

        @pl.when(i == 0)
        def _():
            ocs_ref[...] = jnp.zeros_like(ocs_ref)
            ocss_ref[...] = jnp.zeros_like(ocss_ref)
        ocs_ref[...] += jnp.sum(zo, axis=0, keepdims=True)
        ocss_ref[...] += jnp.sum(zo * zo, axis=0, keepdims=True)

    in_specs = [_row_spec(cols_in)] + [_stat_spec(cols_in)] * 4
    args = [z, cs, css, g.reshape(1, -1), b.reshape(1, -1)]
    if has_idt:
        in_specs.append(_row_spec(cols_in))
        args.append(idt)
    in_specs += [_full_spec(w.shape), _stat_spec(cols_out)]
    args += [w, bias.reshape(1, -1)]
    out_specs = [_row_spec(cols_out), _stat_spec(cols_out),
                 _stat_spec(cols_out)]
    out_shape = [jax.ShapeDtypeStruct((_N, cols_out), jnp.float32),
                 jax.ShapeDtypeStruct((1, cols_out), jnp.float32),
                 jax.ShapeDtypeStruct((1, cols_out), jnp.float32)]
    return pl.pallas_call(
        body, grid=(_GRID,), in_specs=in_specs, out_specs=out_specs,
        out_shape=out_shape)(*args)


def _final_call(z, cs, css, g, b, w, bias):
    cols_in = z.shape[1]

    def body(z_ref, cs_ref, css_ref, g_ref, b_ref, w_ref, bias_ref, o_ref):
        hnew = _bn_relu(z_ref[...], cs_ref[...], css_ref[...],
                        g_ref[...], b_ref[...], None)
        z3 = (jnp.dot(hnew, w_ref[...], preferred_element_type=jnp.float32)
              + bias_ref[...])
        m = jnp.max(z3, axis=1, keepdims=True)
        s = jnp.log(jnp.sum(jnp.exp(z3 - m), axis=1, keepdims=True))
        o_ref[...] = z3 - m - s

    in_specs = [_row_spec(cols_in)] + [_stat_spec(cols_in)] * 4
    in_specs += [_full_spec(w.shape), _stat_spec(_NUM_CLASSES)]
    return pl.pallas_call(
        body, grid=(_GRID,), in_specs=in_specs,
        out_specs=_row_spec(_NUM_CLASSES),
        out_shape=jax.ShapeDtypeStruct((_N, _NUM_CLASSES), jnp.float32))(
            z, cs, css, g.reshape(1, -1), b.reshape(1, -1), w,
            bias.reshape(1, _NUM_CLASSES))


def _emit_tables(xp, a_s_flat, a_d_flat, h, c, asrc_ref, adst_ref):
    rows = xp.shape[0]
    for vec, ref in ((a_s_flat, asrc_ref), (a_d_flat, adst_ref)):
        m = xp * vec
        cols = [jnp.sum(m[:, hh * c:(hh + 1) * c], axis=1, keepdims=True)
                for hh in range(h)]
        cols.append(jnp.zeros((rows, 16 - h), jnp.float32))
        ref[...] = jnp.concatenate(cols, axis=1)


def _run_sc_layer(li, slabs, asrc, adst, aedg, src_p, dst_p):
    fin, c, h, concat = _LAYERS[li]
    hc = h * c
    hc0 = min(hc, 128)
    partials = []
    k0 = _sc_edge_kernel(hc0 + 16, h, c, 0, True)
    partials.append(k0(slabs[0], asrc, adst, aedg, src_p, dst_p))
    if hc > 128:
        k1 = _sc_edge_kernel(128, h, c, 128 // c, False)
        partials.append(k1(slabs[1], asrc, adst, aedg, src_p, dst_p))
    return partials


def kernel(x, edge_index, edge_attr, params):
    src = edge_index[0]
    dst = edge_index[1]
    pad = _EPAD - _E
    src_p = jnp.concatenate([src, jnp.zeros((pad,), jnp.int32)])
    dst_p = jnp.concatenate([dst, jnp.zeros((pad,), jnp.int32)])
    ea_p = jnp.concatenate([edge_attr, jnp.zeros((pad, 3), jnp.float32)])

    we_list = [params[f"conv{li}_We"] for li in range(6)]
    ae_list = [params[f"conv{li}_ae"].reshape(1, -1) for li in range(6)]
    aedg = _aedg_call(ea_p, we_list, ae_list)

    outs = _entry_call(x, params["conv0_W"],
                       params["conv0_as"].reshape(1, 128),
                       params["conv0_ad"].reshape(1, 128), params["res0_W"],
                       params["res0_b"].reshape(1, -1))

    for li in range(6):
        fin, c, h, concat = _LAYERS[li]
        hc = h * c
        nslab = 2 if hc > 128 else 1
        slabs = outs[:nslab]
        asrc, adst = outs[nslab], outs[nslab + 1]
        idt = outs[nslab + 2]
        partials = _run_sc_layer(li, slabs, asrc, adst, aedg[li],
                                 src_p, dst_p)
        out, cs, css = _combine_call(li, partials, params[f"conv{li}_b"])
        g = params[f"bn{li}_g"]
        b = params[f"bn{li}_b"]
        if li < 5:
            outs = _fuse_call(li, out, cs, css, g, b, idt, params)
        else:
            z1, cs1, css1 = _dense_stats_call(
                out, cs, css, g, b, idt, params["mlp_W1"], params["mlp_b1"])

    z2, cs2, css2 = _dense_stats_call(
        z1, cs1, css1, params["mlp_bn1_g"], params["mlp_bn1_b"], None,
        params["mlp_W2"], params["mlp_b2"])
    return _final_call(z2, cs2, css2, params["mlp_bn2_g"],
                       params["mlp_bn2_b"], params["mlp_W3"],
                       params["mlp_b3"])


# speedup vs baseline: 18.4487x; 1.0576x over previous
import functools

import jax
import jax.numpy as jnp
from jax import lax
from jax.experimental import pallas as pl
from jax.experimental.pallas import tpu as pltpu
from jax.experimental.pallas import tpu_sc as plsc

_N = 10000
_E = 320000
_F_IN = 128
_NUM_CLASSES = 5
_LAYERS = [(128, 32, 4, True), (128, 32, 4, True), (128, 64, 4, True),
           (256, 64, 4, True), (256, 32, 4, True), (128, 32, 1, False)]
_RES = [(128, 128), None, (128, 256), None, (256, 128), (128, 32)]

_CH = 64
_NW = 32
_EPAD = 323584
_CPW = _EPAD // (_CH * _NW)
_RB = 1000
_GRID = _N // _RB
_NPAD = 10240
_NPS = _NPAD // 16


@functools.lru_cache(maxsize=None)
def _sc_edge_kernel(w_row, h, c, head_off, has_ones):
    hc0 = w_row - 16 if has_ones else w_row
    nheads = hc0 // c
    nv = w_row // 16

    mesh = plsc.VectorSubcoreMesh(core_axis_name="c", subcore_axis_name="s")

    @functools.partial(
        pl.kernel,
        out_type=jax.ShapeDtypeStruct((2 * _NPAD, w_row), jnp.float32),
        mesh=mesh,
        compiler_params=pltpu.CompilerParams(needs_layout_passes=False,
                                             use_tc_tiling_on_sc=False),
        scratch_types=[
            pltpu.VMEM_SHARED((_NPAD, w_row), jnp.float32),
            pltpu.VMEM((_CH,), jnp.int32),
            pltpu.VMEM((_CH,), jnp.int32),
            pltpu.VMEM((_CH,), jnp.int32),
            pltpu.VMEM((_CH,), jnp.int32),
            pltpu.VMEM((_CH, 16), jnp.float32),
            pltpu.VMEM((_CH, 16), jnp.float32),
            pltpu.VMEM((_CH, 16), jnp.float32),
            pltpu.VMEM((_CH, 16), jnp.float32),
            pltpu.VMEM((_CH, h), jnp.float32),
            pltpu.VMEM((_CH, h), jnp.float32),
            pltpu.VMEM((_CH * h,), jnp.float32),
            pltpu.VMEM((_CH, w_row), jnp.float32),
            pltpu.VMEM((_CH, w_row), jnp.float32),
            pltpu.SemaphoreType.DMA,
            pltpu.SemaphoreType.DMA,
            pltpu.SemaphoreType.DMA,
            pltpu.SemaphoreType.DMA,
            pltpu.SemaphoreType.DMA,
            pltpu.SemaphoreType.DMA,
        ],
    )
    def k(xp_hbm, asrc_hbm, adst_hbm, aedg_hbm, src_hbm, dst_hbm, out_hbm,
          acc, src0, src1, dst0, dst1, asg0, asg1, adg0, adg1, aedg0, aedg1,
          w_v, rows0, rows1, s00, s01, s02, s10, s11, s12):
        cid = lax.axis_index("c")
        sid = lax.axis_index("s")
        wid = cid * 16 + sid
        zeros16 = jnp.zeros((16,), jnp.float32)
        lanes = lax.iota(jnp.int32, 16)
        bufs = [(src0, dst0, asg0, adg0, aedg0, rows0, (s00, s01, s02)),
                (src1, dst1, asg1, adg1, aedg1, rows1, (s10, s11, s12))]

        def zrow(i, carry):
            for j in range(nv):
                rows0[i, pl.ds(16 * j, 16)] = zeros16
            return carry
        lax.fori_loop(0, _CH, zrow, 0)
        for rep in range(_NPS // _CH):
            pltpu.sync_copy(rows0,
                            acc.at[pl.ds(sid * _NPS + rep * _CH, _CH)])
        plsc.subcore_barrier()

        w_lo = 0 if has_ones else head_off
        w_hi = h if has_ones else head_off + nheads

        def issue(buf, t):
            src_v, dst_v, asg_v, adg_v, aedg_v, rows_v, sems = buf
            base = (wid * _CPW + t) * _CH
            pltpu.sync_copy(src_hbm.at[pl.ds(base, _CH)], src_v)
            pltpu.sync_copy(dst_hbm.at[pl.ds(base, _CH)], dst_v)
            pltpu.sync_copy(aedg_hbm.at[pl.ds(base, _CH)], aedg_v)
            return [pltpu.async_copy(asrc_hbm.at[src_v], asg_v, sems[0]),
                    pltpu.async_copy(adst_hbm.at[dst_v], adg_v, sems[1]),
                    pltpu.async_copy(xp_hbm.at[src_v], rows_v, sems[2])]

        def process(buf, cps):
            src_v, dst_v, asg_v, adg_v, aedg_v, rows_v, sems = buf
            for cp in cps:
                cp.wait()
            for g in range(_CH // 16):
                eidx = lanes + (16 * g)
                for hh in range(w_lo, w_hi):
                    col = jnp.full((16,), hh, jnp.int32)
                    al = (plsc.load_gather(asg_v, [eidx, col])
                          + plsc.load_gather(adg_v, [eidx, col])
                          + plsc.load_gather(aedg_v, [eidx, col]))
                    al = jnp.maximum(al, 0.2 * al)
                    plsc.store_scatter(w_v, [eidx * h + hh], jnp.exp(al))

            def scale(e, carry):
                e16 = jnp.full((16,), e * h, jnp.int32)
                for head in range(nheads):
                    wb = plsc.load_gather(w_v, [e16 + (head_off + head)])
                    for j in range(head * c // 16, (head + 1) * c // 16):
                        rows_v[e, pl.ds(16 * j, 16)] = (
                            rows_v[e, pl.ds(16 * j, 16)] * wb)
                if has_ones:
                    wl = plsc.load_gather(
                        w_v, [e16 + jnp.minimum(lanes, h - 1)])
                    rows_v[e, pl.ds(hc0, 16)] = (
                        rows_v[e, pl.ds(hc0, 16)] * wl)
                return carry
            lax.fori_loop(0, _CH, scale, 0)

            pltpu.sync_copy(rows_v, acc.at[dst_v], add=True)

        def pair(t, carry):
            cps0 = issue(bufs[0], 2 * t)
            cps1 = issue(bufs[1], 2 * t + 1)
            process(bufs[0], cps0)
            process(bufs[1], cps1)
            return carry
        lax.fori_loop(0, _CPW // 2, pair, 0)

        plsc.subcore_barrier()
        pltpu.sync_copy(acc.at[pl.ds(sid * _NPS, _NPS)],
                        out_hbm.at[pl.ds(cid * _NPAD + sid * _NPS, _NPS)])

    return k


def _full_spec(shape):
    return pl.BlockSpec(shape, lambda i: tuple(0 for _ in shape))


def _row_spec(cols):
    return pl.BlockSpec((_RB, cols), lambda i: (i, 0))


def _stat_spec(cols):
    return pl.BlockSpec((1, cols), lambda i: (0, 0))


def _aedg_call(edge_attr_pad, we_list, ae_list):
    eb = 2048
    grid = _EPAD // eb
    hs = [1 if w.shape[1] == 32 else 4 for w in we_list]

    def body(ea_ref, *refs):
        wrefs = refs[:6]
        arefs = refs[6:12]
        orefs = refs[12:]
        i = pl.program_id(0)
        row = i * eb + lax.broadcasted_iota(jnp.int32, (eb, 1), 0)
        pad = jnp.where(row >= _E, -1e30, 0.0)
        ea = ea_ref[...]
        for li in range(6):
            hc = wrefs[li].shape[1]
            h = hs[li]
            c = hc // h
            ep = jnp.dot(ea, wrefs[li][...],
                         preferred_element_type=jnp.float32)
            m = ep * arefs[li][...]
            cols = [jnp.sum(m[:, hh * c:(hh + 1) * c], axis=1, keepdims=True)
                    for hh in range(h)]
            a = jnp.concatenate(cols, axis=1) if h > 1 else cols[0]
            orefs[li][...] = a + pad

    in_specs = [pl.BlockSpec((eb, 3), lambda i: (i, 0))]
    in_specs += [_full_spec(w.shape) for w in we_list]
    in_specs += [_stat_spec(a.shape[1]) for a in ae_list]
    out_specs = [pl.BlockSpec((eb, h), lambda i: (i, 0)) for h in hs]
    out_shape = [jax.ShapeDtypeStruct((_EPAD, h), jnp.float32) for h in hs]
    return pl.pallas_call(
        body, grid=(grid,), in_specs=in_specs, out_specs=out_specs,
        out_shape=out_shape)(edge_attr_pad, *we_list, *ae_list)


def _ones_pat(h):
    return (lax.broadcasted_iota(jnp.int32, (1, 16), 1) < h).astype(jnp.float32)


def _emit_slabs(xp, h, c, slab_refs):
    hc = h * c
    hc0 = min(hc, 128)
    pat = jnp.broadcast_to(_ones_pat(h), (_RB, 16))
    slab_refs[0][...] = jnp.concatenate([xp[:, :hc0], pat], axis=1)
    if hc > 128:
        slab_refs[1][...] = xp[:, 128:]


def _prep_outputs(li):
    fin, c, h, concat = _LAYERS[li]
    hc = h * c
    hc0 = min(hc, 128)
    shapes = [jax.ShapeDtypeStruct((_N, hc0 + 16), jnp.float32)]
    specs = [_row_spec(hc0 + 16)]
    if hc > 128:
        shapes.append(jax.ShapeDtypeStruct((_N, 128), jnp.float32))
        specs.append(_row_spec(128))
    shapes += [jax.ShapeDtypeStruct((_N, 16), jnp.float32)] * 2
    specs += [_row_spec(16)] * 2
    fo = _RES[li][1] if _RES[li] is not None else _LAYERS[li][0]
    shapes.append(jax.ShapeDtypeStruct((_N, fo), jnp.float32))
    specs.append(_row_spec(fo))
    return shapes, specs


def _entry_call(x, w0, as0, ad0, res_w, res_b):
    def body(x_ref, w_ref, as_ref, ad_ref, rw_ref, rb_ref,
             slab_ref, asrc_ref, adst_ref, idt_ref):
        xb = x_ref[...]
        xp = jnp.dot(xb, w_ref[...], preferred_element_type=jnp.float32)
        _emit_slabs(xp, 4, 32, [slab_ref])
        _emit_tables(xp, as_ref[...], ad_ref[...], 4, 32, asrc_ref, adst_ref)
        idt_ref[...] = (jnp.dot(xb, rw_ref[...],
                                preferred_element_type=jnp.float32)
                        + rb_ref[...])

    shapes, specs = _prep_outputs(0)
    in_specs = [_row_spec(_F_IN), _full_spec(w0.shape), _stat_spec(128),
                _stat_spec(128), _full_spec(res_w.shape),
                pl.BlockSpec((1, res_b.shape[1]), lambda i: (0, 0))]
    return pl.pallas_call(
        body, grid=(_GRID,), in_specs=in_specs, out_specs=specs,
        out_shape=shapes)(x, w0, as0, ad0, res_w, res_b)


def _combine_call(li, partials, bias):
    fin, c, h, concat = _LAYERS[li]
    hc = h * c
    hc0 = min(hc, 128)

    flat = []
    for p in partials:
        flat += [p[:_N], p[_NPAD:_NPAD + _N]]

    def body(*refs):
        i = pl.program_id(0)
        n_in = len(flat) + 1
        prefs = refs[:len(flat)]
        bias_ref = refs[len(flat)]
        out_ref, cs_ref, css_ref = refs[n_in:]
        m0 = prefs[0][...] + prefs[1][...]
        den = m0[:, hc0:hc0 + h] + 1e-16
        feats = m0[:, :hc0]
        if len(prefs) == 4:
            feats = jnp.concatenate([feats, prefs[2][...] + prefs[3][...]],
                                    axis=1)
        cols = []
        for hh in range(h):
            cols.append(feats[:, hh * c:(hh + 1) * c] / den[:, hh:hh + 1])
        out = jnp.concatenate(cols, axis=1) + bias_ref[...]
        out_ref[...] = out

        @pl.when(i == 0)
        def _():
            cs_ref[...] = jnp.zeros_like(cs_ref)
            css_ref[...] = jnp.zeros_like(css_ref)
        cs_ref[...] += jnp.sum(out, axis=0, keepdims=True)
        css_ref[...] += jnp.sum(out * out, axis=0, keepdims=True)

    in_specs = [_row_spec(p.shape[1]) for p in flat]
    in_specs.append(_stat_spec(hc))
    out_specs = [_row_spec(hc), _stat_spec(hc), _stat_spec(hc)]
    out_shape = [jax.ShapeDtypeStruct((_N, hc), jnp.float32),
                 jax.ShapeDtypeStruct((1, hc), jnp.float32),
                 jax.ShapeDtypeStruct((1, hc), jnp.float32)]
    return pl.pallas_call(
        body, grid=(_GRID,), in_specs=in_specs, out_specs=out_specs,
        out_shape=out_shape)(*flat, bias.reshape(1, hc))


def _bn_relu(z, cs, css, g, b, idt):
    mu = cs * (1.0 / _N)
    var = css * (1.0 / _N) - mu * mu
    bn = g * (z - mu) / jnp.sqrt(var + 1e-5) + b
    if idt is not None:
        bn = bn + idt
    return jnp.maximum(bn, 0.0)


def _fuse_call(li, out, cs, css, g, b, idt, params):
    nl = li + 1
    fin, c, h, concat = _LAYERS[nl]
    hc = h * c
    w_next = params[f"conv{nl}_W"]
    a_s = params[f"conv{nl}_as"].reshape(1, hc)
    a_d = params[f"conv{nl}_ad"].reshape(1, hc)
    res = _RES[nl]
    hc_prev = out.shape[1]
    if res is not None:
        rw = params[f"res{nl}_W"]
        rb = params[f"res{nl}_b"].reshape(1, -1)

    def body(*refs):
        (out_ref, cs_ref, css_ref, g_ref, b_ref, idt_ref, w_ref,
         as_ref, ad_ref) = refs[:9]
        rest = refs[9:]
        if res is not None:
            rw_ref, rb_ref = rest[:2]
            orefs = rest[2:]
        else:
            orefs = rest
        hnew = _bn_relu(out_ref[...], cs_ref[...], css_ref[...],
                        g_ref[...], b_ref[...], idt_ref[...])
        xp = jnp.dot(hnew, w_ref[...], preferred_element_type=jnp.float32)
        nslab = 2 if hc > 128 else 1
        _emit_slabs(xp, h, c, list(orefs[:nslab]))
        _emit_tables(xp, as_ref[...], ad_ref[...], h, c,
                     orefs[nslab], orefs[nslab + 1])
        if res is not None:
            orefs[nslab + 2][...] = (
                jnp.dot(hnew, rw_ref[...], preferred_element_type=jnp.float32)
                + rb_ref[...])
        else:
            orefs[nslab + 2][...] = hnew

    shapes, specs = _prep_outputs(nl)
    in_specs = [_row_spec(hc_prev), _stat_spec(hc_prev), _stat_spec(hc_prev),
                _stat_spec(hc_prev), _stat_spec(hc_prev), _row_spec(hc_prev),
                _full_spec(w_next.shape), _stat_spec(hc), _stat_spec(hc)]
    args = [out, cs, css, g.reshape(1, -1), b.reshape(1, -1), idt,
            w_next, a_s, a_d]
    if res is not None:
        in_specs += [_full_spec(rw.shape), _stat_spec(rb.shape[1])]
        args += [rw, rb]
    return pl.pallas_call(
        body, grid=(_GRID,), in_specs=in_specs, out_specs=specs,
        out_shape=shapes)(*args)


def _dense_stats_call(z, cs, css, g, b, idt, w, bias):
    cols_in = z.shape[1]
    cols_out = w.shape[1]
    has_idt = idt is not None

    def body(*refs):
        i = pl.program_id(0)
        idx = 0
        z_ref = refs[0]; cs_ref = refs[1]; css_ref = refs[2]
        g_ref = refs[3]; b_ref = refs[4]
        idx = 5
        idt_ref = None
        if has_idt:
            idt_ref = refs[idx]; idx += 1
        w_ref = refs[idx]; bias_ref = refs[idx + 1]
        zo_ref, ocs_ref, ocss_ref = refs[idx + 2:]
        hnew = _bn_relu(z_ref[...], cs_ref[...], css_ref[...],
                        g_ref[...], b_ref[...],
                        idt_ref[...] if has_idt else None)
        zo = (jnp.dot(hnew, w_ref[...], preferred_element_type=jnp.float32)
              + bias_ref[...])
        zo_ref[...] = zo

        @pl.when(i == 0)
        def _():
            ocs_ref[...] = jnp.zeros_like(ocs_ref)
            ocss_ref[...] = jnp.zeros_like(ocss_ref)
        ocs_ref[...] += jnp.sum(zo, axis=0, keepdims=True)
        ocss_ref[...] += jnp.sum(zo * zo, axis=0, keepdims=True)

    in_specs = [_row_spec(cols_in)] + [_stat_spec(cols_in)] * 4
    args = [z, cs, css, g.reshape(1, -1), b.reshape(1, -1)]
    if has_idt:
        in_specs.append(_row_spec(cols_in))
        args.append(idt)
    in_specs += [_full_spec(w.shape), _stat_spec(cols_out)]
    args += [w, bias.reshape(1, -1)]
    out_specs = [_row_spec(cols_out), _stat_spec(cols_out),
                 _stat_spec(cols_out)]
    out_shape = [jax.ShapeDtypeStruct((_N, cols_out), jnp.float32),
                 jax.ShapeDtypeStruct((1, cols_out), jnp.float32),
                 jax.ShapeDtypeStruct((1, cols_out), jnp.float32)]
    return pl.pallas_call(
        body, grid=(_GRID,), in_specs=in_specs, out_specs=out_specs,
        out_shape=out_shape)(*args)


def _final_call(z, cs, css, g, b, w, bias):
    cols_in = z.shape[1]

    def body(z_ref, cs_ref, css_ref, g_ref, b_ref, w_ref, bias_ref, o_ref):
        hnew = _bn_relu(z_ref[...], cs_ref[...], css_ref[...],
                        g_ref[...], b_ref[...], None)
        z3 = (jnp.dot(hnew, w_ref[...], preferred_element_type=jnp.float32)
              + bias_ref[...])
        m = jnp.max(z3, axis=1, keepdims=True)
        s = jnp.log(jnp.sum(jnp.exp(z3 - m), axis=1, keepdims=True))
        o_ref[...] = z3 - m - s

    in_specs = [_row_spec(cols_in)] + [_stat_spec(cols_in)] * 4
    in_specs += [_full_spec(w.shape), _stat_spec(_NUM_CLASSES)]
    return pl.pallas_call(
        body, grid=(_GRID,), in_specs=in_specs,
        out_specs=_row_spec(_NUM_CLASSES),
        out_shape=jax.ShapeDtypeStruct((_N, _NUM_CLASSES), jnp.float32))(
            z, cs, css, g.reshape(1, -1), b.reshape(1, -1), w,
            bias.reshape(1, _NUM_CLASSES))


def _emit_tables(xp, a_s_flat, a_d_flat, h, c, asrc_ref, adst_ref):
    rows = xp.shape[0]
    for vec, ref in ((a_s_flat, asrc_ref), (a_d_flat, adst_ref)):
        m = xp * vec
        cols = [jnp.sum(m[:, hh * c:(hh + 1) * c], axis=1, keepdims=True)
                for hh in range(h)]
        cols.append(jnp.zeros((rows, 16 - h), jnp.float32))
        ref[...] = jnp.concatenate(cols, axis=1)


def _run_sc_layer(li, slabs, asrc, adst, aedg, src_p, dst_p):
    fin, c, h, concat = _LAYERS[li]
    hc = h * c
    hc0 = min(hc, 128)
    partials = []
    k0 = _sc_edge_kernel(hc0 + 16, h, c, 0, True)
    partials.append(k0(slabs[0], asrc, adst, aedg, src_p, dst_p))
    if hc > 128:
        k1 = _sc_edge_kernel(128, h, c, 128 // c, False)
        partials.append(k1(slabs[1], asrc, adst, aedg, src_p, dst_p))
    return partials


def kernel(x, edge_index, edge_attr, params):
    src = edge_index[0]
    dst = edge_index[1]
    pad = _EPAD - _E
    src_p = jnp.concatenate([src, jnp.zeros((pad,), jnp.int32)])
    dst_p = jnp.concatenate([dst, jnp.zeros((pad,), jnp.int32)])
    ea_p = jnp.concatenate([edge_attr, jnp.zeros((pad, 3), jnp.float32)])

    we_list = [params[f"conv{li}_We"] for li in range(6)]
    ae_list = [params[f"conv{li}_ae"].reshape(1, -1) for li in range(6)]
    aedg = _aedg_call(ea_p, we_list, ae_list)

    outs = _entry_call(x, params["conv0_W"],
                       params["conv0_as"].reshape(1, 128),
                       params["conv0_ad"].reshape(1, 128), params["res0_W"],
                       params["res0_b"].reshape(1, -1))

    for li in range(6):
        fin, c, h, concat = _LAYERS[li]
        hc = h * c
        nslab = 2 if hc > 128 else 1
        slabs = outs[:nslab]
        asrc, adst = outs[nslab], outs[nslab + 1]
        idt = outs[nslab + 2]
        partials = _run_sc_layer(li, slabs, asrc, adst, aedg[li],
                                 src_p, dst_p)
        out, cs, css = _combine_call(li, partials, params[f"conv{li}_b"])
        g = params[f"bn{li}_g"]
        b = params[f"bn{li}_b"]
        if li < 5:
            outs = _fuse_call(li, out, cs, css, g, b, idt, params)
        else:
            z1, cs1, css1 = _dense_stats_call(
                out, cs, css, g, b, idt, params["mlp_W1"], params["mlp_b1"])

    z2, cs2, css2 = _dense_stats_call(
        z1, cs1, css1, params["mlp_bn1_g"], params["mlp_bn1_b"], None,
        params["mlp_W2"], params["mlp_b2"])
    return _final_call(z2, cs2, css2, params["mlp_bn2_g"],
                       params["mlp_bn2_b"], params["mlp_W3"],
                       params["mlp_b3"])
